# baseline (speedup 1.0000x reference)
import functools

import jax
import jax.numpy as jnp
from jax import lax
from jax.experimental import pallas as pl
from jax.experimental.pallas import tpu as pltpu
from jax.experimental.pallas import tpu_sc as plsc

B_TOK, SEQ = 16384, 50
V, D = 1000000, 64
B = B_TOK * SEQ
NC, NS = 2, 16
NW = NC * NS
B_PER_W = B // NW
CHUNK = 640
SUB = 128
K = CHUNK // SUB
N_CHUNKS = B_PER_W // CHUNK
NBUF = 2
N_GROUPS = N_CHUNKS // NBUF

_mesh = plsc.VectorSubcoreMesh(core_axis_name="c", subcore_axis_name="s")


@functools.partial(
    pl.kernel,
    mesh=_mesh,
    out_type=jax.ShapeDtypeStruct((B, D), jnp.float32),
    scratch_types=[
        pltpu.VMEM((NBUF, CHUNK), jnp.int32),
        pltpu.VMEM((NBUF, CHUNK, D), jnp.float32),
        pltpu.SemaphoreType.DMA((NBUF,)),
        pltpu.SemaphoreType.DMA((NBUF,)),
    ],
    compiler_params=pltpu.CompilerParams(use_tc_tiling_on_sc=False),
)
def _gather_kernel(table_hbm, idx_hbm, out_hbm, idx_v, rows_v, gsem, wsem):
    wid = lax.axis_index("s") * NC + lax.axis_index("c")
    base = wid * B_PER_W

    def fire_gathers(b):
        for j in range(K):
            pltpu.async_copy(
                table_hbm.at[idx_v.at[b, pl.ds(j * SUB, SUB)]],
                rows_v.at[b, pl.ds(j * SUB, SUB)], gsem.at[b])

    def start_gather(ck, b):
        off = base + ck * CHUNK
        pltpu.sync_copy(idx_hbm.at[pl.ds(off, CHUNK)], idx_v.at[b])
        fire_gathers(b)

    for b in range(NBUF):
        start_gather(b, b)

    def body(i, carry):
        for b in range(NBUF):
            ck = i * NBUF + b
            pltpu.make_async_copy(table_hbm.at[idx_v.at[b]], rows_v.at[b],
                                  gsem.at[b]).wait()
            pltpu.async_copy(
                rows_v.at[b], out_hbm.at[pl.ds(base + ck * CHUNK, CHUNK)],
                wsem.at[b])
            nk = ck + NBUF
            off = base + nk * CHUNK
            pltpu.sync_copy(idx_hbm.at[pl.ds(off, CHUNK)], idx_v.at[b])
            pltpu.make_async_copy(
                rows_v.at[b], out_hbm.at[pl.ds(base + ck * CHUNK, CHUNK)],
                wsem.at[b]).wait()
            fire_gathers(b)
        return carry

    lax.fori_loop(0, N_GROUPS - 1, body, 0)

    for b in range(NBUF):
        ck = (N_GROUPS - 1) * NBUF + b
        pltpu.make_async_copy(table_hbm.at[idx_v.at[b]], rows_v.at[b],
                              gsem.at[b]).wait()
        pltpu.async_copy(rows_v.at[b],
                         out_hbm.at[pl.ds(base + ck * CHUNK, CHUNK)],
                         wsem.at[b])
    for b in range(NBUF):
        ck = (N_GROUPS - 1) * NBUF + b
        pltpu.make_async_copy(rows_v.at[b],
                              out_hbm.at[pl.ds(base + ck * CHUNK, CHUNK)],
                              wsem.at[b]).wait()


V_BLK = 4096
N_VBLK = -(-V // V_BLK)
N_BLK = 128
N_NBLK = B_TOK // N_BLK


def _wt_body(wt_ref, o_ref):
    o_ref[...] = wt_ref[...].T


_w_transpose = pl.pallas_call(
    _wt_body,
    grid=(N_VBLK,),
    in_specs=[pl.BlockSpec((D, V_BLK), lambda i: (0, i))],
    out_specs=pl.BlockSpec((V_BLK, D), lambda i: (i, 0)),
    out_shape=jax.ShapeDtypeStruct((V, D), jnp.float32),
)


def _ot_body(r_ref, o_ref):
    x = r_ref[...].reshape(N_BLK, SEQ, D)
    for s in range(SEQ):
        o_ref[s, :, :] = x[:, s, :].T


_o_transpose = pl.pallas_call(
    _ot_body,
    grid=(N_NBLK,),
    in_specs=[pl.BlockSpec((N_BLK * SEQ, D), lambda i: (i, 0))],
    out_specs=pl.BlockSpec((SEQ, D, N_BLK), lambda i: (0, 0, i)),
    out_shape=jax.ShapeDtypeStruct((SEQ, D, B_TOK), jnp.float32),
)


def kernel(token_ids, weights):
    wt = jnp.swapaxes(weights, 0, 1)
    w_rm = _w_transpose(wt)
    flat = token_ids.reshape(-1).astype(jnp.int32)
    raw = _gather_kernel(w_rm, flat)
    out_t = _o_transpose(raw)
    return jnp.transpose(out_t, (2, 0, 1))


# speedup vs baseline: 1.0900x; 1.0900x over previous
import functools

import jax
import jax.numpy as jnp
from jax import lax
from jax.experimental import pallas as pl
from jax.experimental.pallas import tpu as pltpu
from jax.experimental.pallas import tpu_sc as plsc

B_TOK, SEQ = 16384, 50
V, D = 1000000, 64
NC, NS = 2, 16
NW = NC * NS

P = 5
SP = SEQ // P
B_P = B_TOK * SP
B_PER_W = B_P // NW
CHUNK = 640
SUB = 128
K = CHUNK // SUB
N_CHUNKS = B_PER_W // CHUNK
NBUF = 2
N_GROUPS = N_CHUNKS // NBUF

_mesh = plsc.VectorSubcoreMesh(core_axis_name="c", subcore_axis_name="s")


@functools.partial(
    pl.kernel,
    mesh=_mesh,
    out_type=jax.ShapeDtypeStruct((B_P, D), jnp.float32),
    scratch_types=[
        pltpu.VMEM((NBUF, CHUNK), jnp.int32),
        pltpu.VMEM((NBUF, CHUNK, D), jnp.float32),
        pltpu.SemaphoreType.DMA((NBUF,)),
        pltpu.SemaphoreType.DMA((NBUF,)),
    ],
    compiler_params=pltpu.CompilerParams(use_tc_tiling_on_sc=False),
)
def _gather_kernel(table_hbm, idx_hbm, out_hbm, idx_v, rows_v, gsem, wsem):
    wid = lax.axis_index("s") * NC + lax.axis_index("c")
    base = wid * B_PER_W

    def fire_gathers(b):
        for j in range(K):
            pltpu.async_copy(
                table_hbm.at[idx_v.at[b, pl.ds(j * SUB, SUB)]],
                rows_v.at[b, pl.ds(j * SUB, SUB)], gsem.at[b])

    def start_gather(ck, b):
        off = base + ck * CHUNK
        pltpu.sync_copy(idx_hbm.at[pl.ds(off, CHUNK)], idx_v.at[b])
        fire_gathers(b)

    for b in range(NBUF):
        start_gather(b, b)

    def body(i, carry):
        for b in range(NBUF):
            ck = i * NBUF + b
            pltpu.make_async_copy(table_hbm.at[idx_v.at[b]], rows_v.at[b],
                                  gsem.at[b]).wait()
            pltpu.async_copy(
                rows_v.at[b], out_hbm.at[pl.ds(base + ck * CHUNK, CHUNK)],
                wsem.at[b])
            nk = ck + NBUF
            off = base + nk * CHUNK
            pltpu.sync_copy(idx_hbm.at[pl.ds(off, CHUNK)], idx_v.at[b])
            pltpu.make_async_copy(
                rows_v.at[b], out_hbm.at[pl.ds(base + ck * CHUNK, CHUNK)],
                wsem.at[b]).wait()
            fire_gathers(b)
        return carry

    lax.fori_loop(0, N_GROUPS - 1, body, 0)

    for b in range(NBUF):
        ck = (N_GROUPS - 1) * NBUF + b
        pltpu.make_async_copy(table_hbm.at[idx_v.at[b]], rows_v.at[b],
                              gsem.at[b]).wait()
        pltpu.async_copy(rows_v.at[b],
                         out_hbm.at[pl.ds(base + ck * CHUNK, CHUNK)],
                         wsem.at[b])
    for b in range(NBUF):
        ck = (N_GROUPS - 1) * NBUF + b
        pltpu.make_async_copy(rows_v.at[b],
                              out_hbm.at[pl.ds(base + ck * CHUNK, CHUNK)],
                              wsem.at[b]).wait()


V_BLK = 4096
N_VBLK = -(-V // V_BLK)


def _wt_body(wt_ref, o_ref):
    o_ref[...] = wt_ref[...].T


_w_transpose = pl.pallas_call(
    _wt_body,
    grid=(N_VBLK,),
    in_specs=[pl.BlockSpec((D, V_BLK), lambda i: (0, i))],
    out_specs=pl.BlockSpec((V_BLK, D), lambda i: (i, 0)),
    out_shape=jax.ShapeDtypeStruct((V, D), jnp.float32),
)


def _ot_body(r_ref, o_ref):
    o_ref[0, :, :] = r_ref[...].T


def _make_o_transpose(s0, first):
    if first:
        return pl.pallas_call(
            _ot_body,
            grid=(SP,),
            in_specs=[pl.BlockSpec((B_TOK, D), lambda i: (i, 0))],
            out_specs=pl.BlockSpec((1, D, B_TOK), lambda i: (s0 + i, 0, 0)),
            out_shape=jax.ShapeDtypeStruct((SEQ, D, B_TOK), jnp.float32),
        )

    def body(r_ref, acc_ref, o_ref):
        _ot_body(r_ref, o_ref)

    return pl.pallas_call(
        body,
        grid=(SP,),
        in_specs=[
            pl.BlockSpec((B_TOK, D), lambda i: (i, 0)),
            pl.BlockSpec(memory_space=pl.ANY),
        ],
        out_specs=pl.BlockSpec((1, D, B_TOK), lambda i: (s0 + i, 0, 0)),
        out_shape=jax.ShapeDtypeStruct((SEQ, D, B_TOK), jnp.float32),
        input_output_aliases={1: 0},
    )


_o_transposes = [_make_o_transpose(p * SP, p == 0) for p in range(P)]


def kernel(token_ids, weights):
    wt = jnp.swapaxes(weights, 0, 1)
    w_rm = _w_transpose(wt)
    tt = jnp.swapaxes(token_ids, 0, 1)
    acc = None
    for p in range(P):
        flat_p = tt[p * SP:(p + 1) * SP].reshape(-1)
        raw_p = _gather_kernel(w_rm, flat_p)
        if p == 0:
            acc = _o_transposes[p](raw_p)
        else:
            acc = _o_transposes[p](raw_p, acc)
    return jnp.transpose(acc, (2, 0, 1))


# speedup vs baseline: 1.1719x; 1.0752x over previous
import functools

import jax
import jax.numpy as jnp
from jax import lax
from jax.experimental import pallas as pl
from jax.experimental.pallas import tpu as pltpu
from jax.experimental.pallas import tpu_sc as plsc

B_TOK, SEQ = 16384, 50
V, D = 1000000, 64
NC, NS = 2, 16
NW = NC * NS

P = 5
SP = SEQ // P
B_P = B_TOK * SP
B_PER_W = B_P // NW
CHUNK = 640
SUB = 128
K = CHUNK // SUB
N_CHUNKS = B_PER_W // CHUNK
NBUF = 2
N_GROUPS = N_CHUNKS // NBUF

_mesh = plsc.VectorSubcoreMesh(core_axis_name="c", subcore_axis_name="s")


@functools.partial(
    pl.kernel,
    mesh=_mesh,
    out_type=jax.ShapeDtypeStruct((B_P, D), jnp.float32),
    scratch_types=[
        pltpu.VMEM((NBUF, CHUNK), jnp.int32),
        pltpu.VMEM((NBUF, CHUNK, D), jnp.float32),
        pltpu.SemaphoreType.DMA((NBUF,)),
        pltpu.SemaphoreType.DMA((NBUF,)),
    ],
    compiler_params=pltpu.CompilerParams(use_tc_tiling_on_sc=False),
)
def _gather_kernel(table_hbm, idx_hbm, out_hbm, idx_v, rows_v, gsem, wsem):
    wid = lax.axis_index("s") * NC + lax.axis_index("c")
    base = wid * B_PER_W

    def fire_gathers(b):
        for j in range(K):
            pltpu.async_copy(
                table_hbm.at[idx_v.at[b, pl.ds(j * SUB, SUB)]],
                rows_v.at[b, pl.ds(j * SUB, SUB)], gsem.at[b])

    def start_gather(ck, b):
        off = base + ck * CHUNK
        pltpu.sync_copy(idx_hbm.at[pl.ds(off, CHUNK)], idx_v.at[b])
        fire_gathers(b)

    for b in range(NBUF):
        start_gather(b, b)

    def body(i, carry):
        for b in range(NBUF):
            ck = i * NBUF + b
            pltpu.make_async_copy(table_hbm.at[idx_v.at[b]], rows_v.at[b],
                                  gsem.at[b]).wait()
            pltpu.async_copy(
                rows_v.at[b], out_hbm.at[pl.ds(base + ck * CHUNK, CHUNK)],
                wsem.at[b])
            nk = ck + NBUF
            off = base + nk * CHUNK
            pltpu.sync_copy(idx_hbm.at[pl.ds(off, CHUNK)], idx_v.at[b])
            pltpu.make_async_copy(
                rows_v.at[b], out_hbm.at[pl.ds(base + ck * CHUNK, CHUNK)],
                wsem.at[b]).wait()
            fire_gathers(b)
        return carry

    lax.fori_loop(0, N_GROUPS - 1, body, 0)

    for b in range(NBUF):
        ck = (N_GROUPS - 1) * NBUF + b
        pltpu.make_async_copy(table_hbm.at[idx_v.at[b]], rows_v.at[b],
                              gsem.at[b]).wait()
        pltpu.async_copy(rows_v.at[b],
                         out_hbm.at[pl.ds(base + ck * CHUNK, CHUNK)],
                         wsem.at[b])
    for b in range(NBUF):
        ck = (N_GROUPS - 1) * NBUF + b
        pltpu.make_async_copy(rows_v.at[b],
                              out_hbm.at[pl.ds(base + ck * CHUNK, CHUNK)],
                              wsem.at[b]).wait()


V_BLK = 32768
N_VBLK = -(-V // V_BLK)


def _wt_body(wt_ref, o_ref):
    o_ref[...] = wt_ref[...].T


_w_transpose = pl.pallas_call(
    _wt_body,
    grid=(N_VBLK,),
    in_specs=[pl.BlockSpec((D, V_BLK), lambda i: (0, i))],
    out_specs=pl.BlockSpec((V_BLK, D), lambda i: (i, 0)),
    out_shape=jax.ShapeDtypeStruct((V, D), jnp.float32),
)


def _ot_body(r_ref, o_ref):
    o_ref[0, :, :] = r_ref[...].T


def _make_o_transpose(s0, first):
    if first:
        return pl.pallas_call(
            _ot_body,
            grid=(SP,),
            in_specs=[pl.BlockSpec((B_TOK, D), lambda i: (i, 0))],
            out_specs=pl.BlockSpec((1, D, B_TOK), lambda i: (s0 + i, 0, 0)),
            out_shape=jax.ShapeDtypeStruct((SEQ, D, B_TOK), jnp.float32),
        )

    def body(r_ref, acc_ref, o_ref):
        _ot_body(r_ref, o_ref)

    return pl.pallas_call(
        body,
        grid=(SP,),
        in_specs=[
            pl.BlockSpec((B_TOK, D), lambda i: (i, 0)),
            pl.BlockSpec(memory_space=pl.ANY),
        ],
        out_specs=pl.BlockSpec((1, D, B_TOK), lambda i: (s0 + i, 0, 0)),
        out_shape=jax.ShapeDtypeStruct((SEQ, D, B_TOK), jnp.float32),
        input_output_aliases={1: 0},
    )


_o_transposes = [_make_o_transpose(p * SP, p == 0) for p in range(P)]


def kernel(token_ids, weights):
    wt = jnp.swapaxes(weights, 0, 1)
    w_rm = _w_transpose(wt)
    tt = jnp.swapaxes(token_ids, 0, 1)
    acc = None
    for p in range(P):
        flat_p = tt[p * SP:(p + 1) * SP].reshape(-1)
        raw_p = _gather_kernel(w_rm, flat_p)
        if p == 0:
            acc = _o_transposes[p](raw_p)
        else:
            acc = _o_transposes[p](raw_p, acc)
    return jnp.transpose(acc, (2, 0, 1))
